# baseline (speedup 1.0000x reference)
import jax
import jax.numpy as jnp
from jax import lax
from jax.experimental import pallas as pl
from jax.experimental.pallas import tpu as pltpu
from jax.experimental.pallas import tpu_sc as plsc

VOCAB_SIZE = 1000000
EMBED_DIM = 64
BATCH_SIZE = 4096
SEQ_LEN = 200

NUM_CORES = 2
NUM_SUBCORES = 16
NUM_WORKERS = NUM_CORES * NUM_SUBCORES
LANES = 16

SEQS_PER_WORKER = BATCH_SIZE // NUM_WORKERS
CHUNK_SEQS = 2
CHUNK_ROWS = CHUNK_SEQS * SEQ_LEN
N_CHUNKS = SEQS_PER_WORKER // CHUNK_SEQS
ROWS_PER_WORKER = SEQS_PER_WORKER * SEQ_LEN

IDX_MINOR = 100
IDX_ROWS_PER_CHUNK = CHUNK_ROWS // IDX_MINOR
D_CHUNKS = EMBED_DIM // LANES


def _positions(seq_len, hidden_size, max_wavelength=10000.0):
    position = jnp.arange(seq_len, dtype=jnp.float32)
    min_freq = 1.0 / max_wavelength
    timescales = jnp.power(
        min_freq,
        (2.0 * (jnp.arange(hidden_size) // 2).astype(jnp.float32))
        / float(hidden_size),
    )
    angles = position[:, None] * timescales[None, :]
    cos_mask = (jnp.arange(hidden_size) % 2).astype(jnp.float32)
    sin_mask = 1.0 - cos_mask
    return jnp.sin(angles) * sin_mask + jnp.cos(angles) * cos_mask


def _sc_kernel(table_hbm, idx_hbm, pos_hbm, out_hbm, idx_v, rows_v, pos_v, sem):
    wid = lax.axis_index("s") * NUM_CORES + lax.axis_index("c")
    row_base = wid * ROWS_PER_WORKER
    idx_row_base = wid * (ROWS_PER_WORKER // IDX_MINOR)

    pltpu.sync_copy(pos_hbm, pos_v)

    @pl.loop(0, N_CHUNKS)
    def _chunk(c):
        row_off = row_base + c * CHUNK_ROWS
        idx_row_off = idx_row_base + c * IDX_ROWS_PER_CHUNK
        pltpu.sync_copy(idx_hbm.at[pl.ds(idx_row_off, IDX_ROWS_PER_CHUNK)], idx_v)
        for j in range(IDX_ROWS_PER_CHUNK):
            pltpu.sync_copy(
                table_hbm.at[idx_v.at[j]],
                rows_v.at[pl.ds(j * IDX_MINOR, IDX_MINOR)],
            )

        @pl.loop(0, SEQ_LEN)
        def _pos_add(l):
            for dd in range(D_CHUNKS):
                pv = pos_v[l, pl.ds(dd * LANES, LANES)]
                for s in range(CHUNK_SEQS):
                    plsc.addupdate(
                        rows_v.at[s * SEQ_LEN + l, pl.ds(dd * LANES, LANES)], pv
                    )

        pltpu.sync_copy(rows_v, out_hbm.at[pl.ds(row_off, CHUNK_ROWS)])


def kernel(x, table):
    idx = x.reshape(BATCH_SIZE * SEQ_LEN // IDX_MINOR, IDX_MINOR).astype(jnp.int32)
    pos = _positions(SEQ_LEN, EMBED_DIM)
    mesh = plsc.VectorSubcoreMesh(core_axis_name="c", subcore_axis_name="s")
    flat = pl.kernel(
        _sc_kernel,
        out_type=jax.ShapeDtypeStruct((BATCH_SIZE * SEQ_LEN, EMBED_DIM), jnp.float32),
        mesh=mesh,
        scratch_types=[
            pltpu.VMEM((IDX_ROWS_PER_CHUNK, IDX_MINOR), jnp.int32),
            pltpu.VMEM((CHUNK_ROWS, EMBED_DIM), jnp.float32),
            pltpu.VMEM((SEQ_LEN, EMBED_DIM), jnp.float32),
            pltpu.SemaphoreType.DMA,
        ],
        compiler_params=pltpu.CompilerParams(use_tc_tiling_on_sc=False),
    )(table, idx, pos)
    return flat.reshape(BATCH_SIZE, SEQ_LEN, EMBED_DIM)


# speedup vs baseline: 1.2206x; 1.2206x over previous
import jax
import jax.numpy as jnp
from jax import lax
from jax.experimental import pallas as pl
from jax.experimental.pallas import tpu as pltpu
from jax.experimental.pallas import tpu_sc as plsc

VOCAB_SIZE = 1000000
EMBED_DIM = 64
BATCH_SIZE = 4096
SEQ_LEN = 200

NUM_CORES = 2
NUM_SUBCORES = 16
NUM_WORKERS = NUM_CORES * NUM_SUBCORES
LANES = 16

SEQS_PER_WORKER = BATCH_SIZE // NUM_WORKERS
CHUNK_SEQS = 2
CHUNK_ROWS = CHUNK_SEQS * SEQ_LEN
N_CHUNKS = SEQS_PER_WORKER // CHUNK_SEQS
ROWS_PER_WORKER = SEQS_PER_WORKER * SEQ_LEN

IDX_MINOR = 100
IDX_ROWS_PER_CHUNK = CHUNK_ROWS // IDX_MINOR
D_CHUNKS = EMBED_DIM // LANES

NBUF = 4


def _positions(seq_len, hidden_size, max_wavelength=10000.0):
    position = jnp.arange(seq_len, dtype=jnp.float32)
    min_freq = 1.0 / max_wavelength
    timescales = jnp.power(
        min_freq,
        (2.0 * (jnp.arange(hidden_size) // 2).astype(jnp.float32))
        / float(hidden_size),
    )
    angles = position[:, None] * timescales[None, :]
    cos_mask = (jnp.arange(hidden_size) % 2).astype(jnp.float32)
    sin_mask = 1.0 - cos_mask
    return jnp.sin(angles) * sin_mask + jnp.cos(angles) * cos_mask


def _sc_kernel(table_hbm, idx_hbm, pos_hbm, out_hbm, *scratch):
    idx_v = scratch[0:NBUF]
    rows_v = scratch[NBUF : 2 * NBUF]
    pos_v = scratch[2 * NBUF]
    sem_i = scratch[2 * NBUF + 1 : 2 * NBUF + 1 + NBUF]
    sem_g = scratch[2 * NBUF + 1 + NBUF : 2 * NBUF + 1 + 2 * NBUF]
    sem_w = scratch[2 * NBUF + 1 + 2 * NBUF : 2 * NBUF + 1 + 3 * NBUF]

    wid = lax.axis_index("s") * NUM_CORES + lax.axis_index("c")
    row_base = wid * ROWS_PER_WORKER
    idx_row_base = wid * (ROWS_PER_WORKER // IDX_MINOR)

    def fire_idx(c, b):
        off = idx_row_base + c * IDX_ROWS_PER_CHUNK
        pltpu.async_copy(
            idx_hbm.at[pl.ds(off, IDX_ROWS_PER_CHUNK)], idx_v[b], sem_i[b]
        )

    def wait_idx(b):
        pltpu.make_async_copy(
            idx_hbm.at[pl.ds(0, IDX_ROWS_PER_CHUNK)], idx_v[b], sem_i[b]
        ).wait()

    def fire_gather(b):
        for j in range(IDX_ROWS_PER_CHUNK):
            pltpu.async_copy(
                table_hbm.at[idx_v[b].at[j]],
                rows_v[b].at[pl.ds(j * IDX_MINOR, IDX_MINOR)],
                sem_g[b],
            )

    def wait_gather(b):
        for j in range(IDX_ROWS_PER_CHUNK):
            pltpu.make_async_copy(
                table_hbm.at[idx_v[b].at[j]],
                rows_v[b].at[pl.ds(j * IDX_MINOR, IDX_MINOR)],
                sem_g[b],
            ).wait()

    def fire_wb(c, b):
        off = row_base + c * CHUNK_ROWS
        pltpu.async_copy(rows_v[b], out_hbm.at[pl.ds(off, CHUNK_ROWS)], sem_w[b])

    def wait_wb(b):
        pltpu.make_async_copy(
            rows_v[b], out_hbm.at[pl.ds(0, CHUNK_ROWS)], sem_w[b]
        ).wait()

    def pos_add(b):
        @pl.loop(0, SEQ_LEN)
        def _pos_add(l):
            for dd in range(D_CHUNKS):
                pv = pos_v[l, pl.ds(dd * LANES, LANES)]
                for s in range(CHUNK_SEQS):
                    plsc.addupdate(
                        rows_v[b].at[s * SEQ_LEN + l, pl.ds(dd * LANES, LANES)], pv
                    )

    pltpu.sync_copy(pos_hbm, pos_v)
    for b in range(NBUF):
        fire_idx(b, b)
    wait_idx(0)
    fire_gather(0)
    wait_idx(1)
    fire_gather(1)

    @pl.loop(0, N_CHUNKS // NBUF)
    def _group(i):
        for b in range(NBUF):
            c = i * NBUF + b
            wait_gather(b)

            @pl.when(i < N_CHUNKS // NBUF - 1)
            def _():
                fire_idx(c + NBUF, b)

            pos_add(b)
            fire_wb(c, b)

            b2 = (b + 2) % NBUF
            if b < 2:
                @pl.when(i >= 1)
                def _():
                    wait_wb(b2)

                wait_idx(b2)
                fire_gather(b2)
            else:
                wait_wb(b2)

                @pl.when(i < N_CHUNKS // NBUF - 1)
                def _():
                    wait_idx(b2)
                    fire_gather(b2)

    wait_wb(2)
    wait_wb(3)


def kernel(x, table):
    idx = x.reshape(BATCH_SIZE * SEQ_LEN // IDX_MINOR, IDX_MINOR).astype(jnp.int32)
    pos = _positions(SEQ_LEN, EMBED_DIM)
    mesh = plsc.VectorSubcoreMesh(core_axis_name="c", subcore_axis_name="s")
    scratch_types = (
        [pltpu.VMEM((IDX_ROWS_PER_CHUNK, IDX_MINOR), jnp.int32)] * NBUF
        + [pltpu.VMEM((CHUNK_ROWS, EMBED_DIM), jnp.float32)] * NBUF
        + [pltpu.VMEM((SEQ_LEN, EMBED_DIM), jnp.float32)]
        + [pltpu.SemaphoreType.DMA] * (3 * NBUF)
    )
    flat = pl.kernel(
        _sc_kernel,
        out_type=jax.ShapeDtypeStruct((BATCH_SIZE * SEQ_LEN, EMBED_DIM), jnp.float32),
        mesh=mesh,
        scratch_types=scratch_types,
        compiler_params=pltpu.CompilerParams(use_tc_tiling_on_sc=False),
    )(table, idx, pos)
    return flat.reshape(BATCH_SIZE, SEQ_LEN, EMBED_DIM)
